# baseline (speedup 1.0000x reference)
import functools

import jax
import jax.numpy as jnp
from jax import lax
from jax.experimental import pallas as pl
from jax.experimental.pallas import tpu as pltpu
from jax.experimental.pallas import tpu_sc as plsc

_D = 28
_DP = 32
_TOTAL = 4096 * 200
_NW = 32
_PER_W = _TOTAL // _NW
_CHUNK = 128
_NCHUNK = _PER_W // _CHUNK
_NBUF = 4
_LEAD = 2


def _sc_gather(table_pad, idx2d):
    mesh = plsc.VectorSubcoreMesh(core_axis_name="c", subcore_axis_name="s")

    @functools.partial(
        pl.kernel,
        mesh=mesh,
        out_type=jax.ShapeDtypeStruct((_TOTAL, _DP), jnp.float32),
        scratch_types=[
            pltpu.VMEM((_NCHUNK, _CHUNK), jnp.int32),
            pltpu.VMEM((_NBUF, _CHUNK, _DP), jnp.float32),
            pltpu.SemaphoreType.DMA,
            pltpu.SemaphoreType.DMA,
            pltpu.SemaphoreType.DMA,
            pltpu.SemaphoreType.DMA,
            pltpu.SemaphoreType.DMA,
            pltpu.SemaphoreType.DMA,
            pltpu.SemaphoreType.DMA,
            pltpu.SemaphoreType.DMA,
        ],
        compiler_params=pltpu.CompilerParams(use_tc_tiling_on_sc=False),
    )
    def k(tbl_hbm, idx_hbm, out_hbm, idx_v, land_v, g0, g1, g2, g3,
          w0, w1, w2, w3):
        gsem = (g0, g1, g2, g3)
        wsem = (w0, w1, w2, w3)
        wid = lax.axis_index("s") * 2 + lax.axis_index("c")
        base = wid * _PER_W
        pltpu.sync_copy(idx_hbm.at[pl.ds(wid * _NCHUNK, _NCHUNK)], idx_v)

        def fire_gather(b, c):
            pltpu.async_copy(tbl_hbm.at[idx_v.at[c]], land_v.at[b], gsem[b])

        def fire_write(b, c):
            pltpu.async_copy(
                land_v.at[b],
                out_hbm.at[pl.ds(base + c * _CHUNK, _CHUNK)],
                wsem[b],
            )

        def wait_gather(b):
            pltpu.make_async_copy(
                out_hbm.at[pl.ds(0, _CHUNK)], land_v.at[b], gsem[b]
            ).wait()

        def wait_write(b):
            pltpu.make_async_copy(
                land_v.at[b], out_hbm.at[pl.ds(0, _CHUNK)], wsem[b]
            ).wait()

        for c0 in range(_LEAD):
            fire_gather(c0 % _NBUF, c0)

        def body(c4, carry):
            for boff in range(_NBUF):
                g = c4 * _NBUF + boff
                blead = (boff + _LEAD) % _NBUF

                @pl.when(g + _LEAD < _NCHUNK)
                def _():
                    @pl.when(g + _LEAD >= _NBUF)
                    def _():
                        wait_write(blead)
                    fire_gather(blead, g + _LEAD)

                wait_gather(boff)
                fire_write(boff, g)
            return carry

        lax.fori_loop(0, _NCHUNK // _NBUF, body, 0)
        for b in range(_NBUF):
            wait_write(b)

    return k(table_pad, idx2d)


def kernel(table, indices):
    idx = indices.astype(jnp.int32).reshape(_TOTAL // _CHUNK, _CHUNK)
    table_pad = (
        jnp.pad(table.reshape(-1, 4, _D), ((0, 0), (0, 0), (0, _DP - _D)))
        .reshape(-1, 4 * _DP)
        .reshape(-1, _DP)
    )
    land = _sc_gather(table_pad, idx)
    return land[:, :_D].reshape(indices.shape[0], indices.shape[1], _D)
